# baseline (speedup 1.0000x reference)
import functools

import jax
import jax.numpy as jnp
from jax.experimental import pallas as pl
from jax.experimental.pallas import tpu as pltpu

N_NODES = 100000
N_DIMS = 768
B = 4
P = 256
K = 64
Q = B * P

NBLK = 2048
NPAD = 100352
NBLOCKS = NPAD // NBLK


def _mm_kernel(pos_ref, keys_ref, out_ref):
    j = pl.program_id(0)
    keys = keys_ref[...]
    nrm2 = jnp.sum(keys * keys, axis=1, keepdims=True)
    recip = jax.lax.rsqrt(jnp.maximum(nrm2, 1e-24))
    keysn = keys * recip
    scores = jax.lax.dot_general(
        pos_ref[...], keysn,
        dimension_numbers=(((1,), (1,)), ((), ())),
        preferred_element_type=jnp.float32,
    )
    col = j * NBLK + jax.lax.broadcasted_iota(jnp.int32, (Q, NBLK), 1)
    out_ref[...] = jnp.where(col < N_NODES, scores, -1e30)


def _scores(positions, keys_raw):
    pos = positions.reshape(Q, N_DIMS)
    keys_pad = jnp.pad(keys_raw, ((0, NPAD - N_NODES), (0, 0)))
    return pl.pallas_call(
        _mm_kernel,
        grid=(NBLOCKS,),
        in_specs=[
            pl.BlockSpec((Q, N_DIMS), lambda j: (0, 0)),
            pl.BlockSpec((NBLK, N_DIMS), lambda j: (j, 0)),
        ],
        out_specs=pl.BlockSpec((Q, NBLK), lambda j: (0, j)),
        out_shape=jax.ShapeDtypeStruct((Q, NPAD), jnp.float32),
    )(pos, keys_pad)


def kernel(positions, k, keys_raw):
    scores = _scores(positions, keys_raw)
    values, indices = jax.lax.top_k(scores.reshape(B, P, NPAD), K)
    values = values + jnp.zeros((), dtype=values.dtype) * k
    return values, indices


# speedup vs baseline: 79.2493x; 79.2493x over previous
import functools

import jax
import jax.numpy as jnp
from jax import lax
from jax.experimental import pallas as pl
from jax.experimental.pallas import tpu as pltpu
from jax.experimental.pallas import tpu_sc as plsc

N_NODES = 100000
N_DIMS = 768
B = 4
P = 256
K = 64
Q = B * P

NBLK = 2048
NPAD = 100352
NBLOCKS = NPAD // NBLK
G = NPAD // 128
GPB = NBLK // 128

NEG = -3.0e38
PADVAL = -1.0e30
CMAX = 256


def _mm_kernel(pos_ref, keys_ref, out_ref, gmax_ref):
    j = pl.program_id(0)
    keys = keys_ref[...]
    nrm2 = jnp.sum(keys * keys, axis=1, keepdims=True)
    recip = lax.rsqrt(jnp.maximum(nrm2, 1e-24))
    keysn = keys * recip
    scores = lax.dot_general(
        pos_ref[...], keysn,
        dimension_numbers=(((1,), (1,)), ((), ())),
        preferred_element_type=jnp.float32,
    )
    col = j * NBLK + lax.broadcasted_iota(jnp.int32, (Q, NBLK), 1)
    scores = jnp.where(col < N_NODES, scores, PADVAL)
    s3 = scores.reshape(Q, GPB, 128)
    out_ref[...] = s3
    gm = jnp.max(s3, axis=2)
    gmax_ref[...] = jnp.pad(gm, ((0, 0), (0, 128 - GPB)),
                            constant_values=NEG)


def _scores(positions, keys_raw):
    pos = positions.reshape(Q, N_DIMS)
    keys_pad = jnp.pad(keys_raw, ((0, NPAD - N_NODES), (0, 0)))
    return pl.pallas_call(
        _mm_kernel,
        grid=(NBLOCKS,),
        in_specs=[
            pl.BlockSpec((Q, N_DIMS), lambda j: (0, 0)),
            pl.BlockSpec((NBLK, N_DIMS), lambda j: (j, 0)),
        ],
        out_specs=[
            pl.BlockSpec((Q, GPB, 128), lambda j: (0, j, 0)),
            pl.BlockSpec((Q, 128), lambda j: (0, j)),
        ],
        out_shape=[
            jax.ShapeDtypeStruct((Q, G, 128), jnp.float32),
            jax.ShapeDtypeStruct((Q, NBLOCKS * 128), jnp.float32),
        ],
    )(pos, keys_pad)


_IOTA = functools.partial(lax.iota, jnp.int32, 16)


def _merge16(tk, ti, ck, ci):
    rk = lax.rev(ck, (0,))
    ri = lax.rev(ci, (0,))
    take = tk >= rk
    hk = jnp.where(take, tk, rk)
    hi = jnp.where(take, ti, ri)
    lk = jnp.where(take, rk, tk)
    li = jnp.where(take, ri, ti)
    hk, hi = plsc.sort_key_val(hk, hi, descending=True)
    lk, li = plsc.sort_key_val(lk, li, descending=True)
    return hk, hi, lk, li


def _insert_block(t8, sk, si):
    ck, ci = sk, si
    out = []
    for t in range(4):
        hk, hi, ck, ci = _merge16(t8[2 * t], t8[2 * t + 1], ck, ci)
        out += [hk, hi]
    return tuple(out)


def _sc_topk_body(scores_ref, gmax_ref, ovals_ref, oidx_ref,
                  gmax_v, gidsel_v, rowids_v, rows_v, cvals_v, cloc_v,
                  stv_v, sti_v, sem):
    nc = 2
    wid = lax.axis_index("s") * nc + lax.axis_index("c")
    qpw = Q // 32

    def qbody(t, _):
        q = wid * qpw + t
        pltpu.sync_copy(gmax_ref.at[q], gmax_v)

        init = []
        for _i in range(4):
            init += [jnp.full((16,), NEG, jnp.float32),
                     jnp.zeros((16,), jnp.int32)]
        init = tuple(init)

        def gb(v, t8):
            x = gmax_v[pl.ds(v * 128, 16)]
            ids = v * 16 + _IOTA()
            sk, si = plsc.sort_key_val(x, ids, descending=True)
            return _insert_block(t8, sk, si)

        t8 = lax.fori_loop(0, G // 16, gb, init)
        tau = t8[6][15]

        for t in range(4):
            gidsel_v[pl.ds(t * 16, 16)] = t8[2 * t + 1]
            rowids_v[pl.ds(t * 16, 16)] = t8[2 * t + 1] + q * G

        pltpu.async_copy(scores_ref.at[rowids_v], rows_v, sem).wait()

        for t in range(CMAX // 16):
            cvals_v[pl.ds(t * 16, 16)] = jnp.full((16,), NEG, jnp.float32)
            cloc_v[pl.ds(t * 16, 16)] = jnp.zeros((16,), jnp.int32)

        def cb(i, cnt):
            row = jnp.full((16,), i, jnp.int32)
            for v8 in range(8):
                x = plsc.load_gather(rows_v, [row, v8 * 16 + _IOTA()])
                m = x >= tau
                loc = i * 128 + v8 * 16 + _IOTA()
                base = jnp.minimum(cnt, CMAX - 16)
                plsc.store_compressed(cvals_v.at[pl.ds(base, 16)], x, mask=m)
                plsc.store_compressed(cloc_v.at[pl.ds(base, 16)], loc, mask=m)
                cnt = cnt + plsc.all_reduce_population_count(m)[0]
            return cnt

        cnt = lax.fori_loop(0, K, cb, jnp.int32(0))
        nblk = (jnp.minimum(cnt, CMAX) + 15) // 16

        def fb(b, t8):
            kx = cvals_v[pl.ds(b * 16, 16)]
            lx = cloc_v[pl.ds(b * 16, 16)]
            g = plsc.load_gather(gidsel_v, [lax.shift_right_logical(lx, 7)])
            absid = g * 128 + (lx & 127)
            sk, si = plsc.sort_key_val(kx, absid, descending=True)
            return _insert_block(t8, sk, si)

        t8f = lax.fori_loop(0, nblk, fb, init)

        for t in range(4):
            stv_v[pl.ds(t * 16, 16)] = t8f[2 * t]
            sti_v[pl.ds(t * 16, 16)] = t8f[2 * t + 1]
        pltpu.sync_copy(stv_v, ovals_ref.at[q])
        pltpu.sync_copy(sti_v, oidx_ref.at[q])
        return 0

    lax.fori_loop(0, qpw, qbody, 0)


_SC_TOPK_OUT = (
    jax.ShapeDtypeStruct((Q, K), jnp.float32),
    jax.ShapeDtypeStruct((Q, K), jnp.int32),
)
_SC_TOPK_SCRATCH = [
    pltpu.VMEM((NBLOCKS * 128,), jnp.float32),
    pltpu.VMEM((K,), jnp.int32),
    pltpu.VMEM((K,), jnp.int32),
    pltpu.VMEM((K, 128), jnp.float32),
    pltpu.VMEM((CMAX,), jnp.float32),
    pltpu.VMEM((CMAX,), jnp.int32),
    pltpu.VMEM((K,), jnp.float32),
    pltpu.VMEM((K,), jnp.int32),
    pltpu.SemaphoreType.DMA,
]

@functools.cache
def _sc_topk():
    return pl.kernel(
        _sc_topk_body,
        out_type=_SC_TOPK_OUT,
        mesh=plsc.VectorSubcoreMesh(core_axis_name="c", subcore_axis_name="s"),
        scratch_types=_SC_TOPK_SCRATCH,
        compiler_params=pltpu.CompilerParams(needs_layout_passes=False),
    )


def kernel(positions, k, keys_raw):
    scores3, gmax = _scores(positions, keys_raw)
    scores2 = scores3.reshape(Q * G, 128)
    vals, idx = _sc_topk()(scores2, gmax)
    values = vals.reshape(B, P, K)
    indices = idx.reshape(B, P, K)
    values = values + jnp.zeros((), dtype=values.dtype) * k
    return values, indices
